# baseline (speedup 1.0000x reference)
import functools

import jax
import jax.numpy as jnp
import numpy as np
from jax import lax
from jax.experimental import pallas as pl
from jax.experimental.pallas import tpu as pltpu
from jax.experimental.pallas import tpu_sc as plsc

_FEATS = ["car_num", "queue_length", "occupancy", "flow", "stop_car_num", "pressure"]
_BS = 50000
_L = 7
_H = 256
_NB = 4
_NH = 4
_HD = 64
_BLK = 2000

_NW = 32
_CHUNK = 128
_ROWS1 = 24000
_ROWS2 = 26000


def _enc_perm() -> np.ndarray:
    perm = np.empty(224, np.int32)
    for j in range(4):
        for g in range(6):
            for l in range(_L):
                perm[j * 42 + g * 7 + l] = l * 32 + g * 4 + j
    for j in range(4):
        for l in range(_L):
            perm[168 + j * 7 + l] = l * 32 + 24 + j
            perm[196 + j * 7 + l] = l * 32 + 28 + j
    return perm


def _kernel_a(f0_ref, f1_ref, f2_ref, f3_ref, f4_ref, f5_ref,
              ph_ref, mk_ref, hid_ref,
              wrep_ref, brep_ref, pe_ref, me_ref,
              wfc1_ref, bfc1_ref, wih_ref, bih_ref, whh_ref, bhh_ref,
              wnh_ref,
              h_out, u_out):
    f32 = jnp.float32
    sig = jax.nn.sigmoid
    xall = jnp.concatenate(
        [f0_ref[...], f1_ref[...], f2_ref[...], f3_ref[...],
         f4_ref[...], f5_ref[...]], axis=1)
    ph = ph_ref[...]
    mk = mk_ref[...]
    parts = []
    for j in range(4):
        parts.append(xall * wrep_ref[j:j + 1, :] + brep_ref[j:j + 1, :])
    for j in range(4):
        parts.append(jnp.where(ph == 0, pe_ref[0, j], pe_ref[1, j]))
    for j in range(4):
        parts.append(jnp.where(mk == 0, me_ref[0, j], me_ref[1, j]))
    bf16 = jnp.bfloat16
    enc = sig(jnp.concatenate(parts, axis=1))
    x = jnp.maximum(
        jnp.dot(enc.astype(bf16), wfc1_ref[...], preferred_element_type=f32)
        + bfc1_ref[...], 0.0)
    hid = hid_ref[...]
    gi = jnp.dot(x.astype(bf16), wih_ref[...],
                 preferred_element_type=f32) + bih_ref[...]
    gh = jnp.dot(hid.astype(bf16), whh_ref[...],
                 preferred_element_type=f32) + bhh_ref[...]
    r = sig(gi[:, :_H] + gh[:, :_H])
    z = sig(gi[:, _H:2 * _H] + gh[:, _H:2 * _H])
    n = jnp.tanh(gi[:, 2 * _H:] + r * gh[:, 2 * _H:])
    h = (1.0 - z) * n + z * hid
    h16 = h.astype(bf16)
    h_out[...] = h
    u = jnp.dot(h16, wnh_ref[...], preferred_element_type=f32)
    lo = lax.bitcast_convert_type(
        u[:, :128].astype(jnp.bfloat16).astype(f32), jnp.int32)
    hi = lax.bitcast_convert_type(
        u[:, 128:].astype(jnp.bfloat16).astype(f32), jnp.int32)
    u_out[...] = lax.bitwise_or(
        lax.shift_right_logical(lo, 16), lax.bitwise_and(hi, jnp.int32(-65536)))


def _sc_gather(table, idx_pad, per_w):
    mesh = plsc.VectorSubcoreMesh(core_axis_name="c", subcore_axis_name="s")

    @functools.partial(
        pl.kernel,
        out_type=jax.ShapeDtypeStruct((_NW * per_w, 128), jnp.int32),
        mesh=mesh,
        scratch_types=[
            pltpu.VMEM((per_w,), jnp.int32),
            pltpu.VMEM((_CHUNK, 128), jnp.int32),
            pltpu.VMEM((_CHUNK, 128), jnp.int32),
            pltpu.SemaphoreType.DMA,
            pltpu.SemaphoreType.DMA,
        ],
    )
    def gather_k(table_hbm, idx_hbm, out_hbm, idx_v, rows0, rows1, sem0, sem1):
        wid = lax.axis_index("s") * 2 + lax.axis_index("c")
        base = pl.multiple_of(wid * per_w, 8)
        pltpu.sync_copy(idx_hbm.at[pl.ds(base, per_w)], idx_v)
        nch = per_w // _CHUNK

        def gstart(c, buf, sem):
            off = pl.multiple_of(c * _CHUNK, 8)
            pltpu.async_copy(table_hbm.at[idx_v.at[pl.ds(off, _CHUNK)]], buf, sem)

        def gwait(buf, sem):
            pltpu.make_async_copy(
                table_hbm.at[idx_v.at[pl.ds(0, _CHUNK)]], buf, sem).wait()

        def wout(c, buf):
            off = pl.multiple_of(c * _CHUNK, 8)
            pltpu.sync_copy(buf, out_hbm.at[pl.ds(base + off, _CHUNK)])

        gstart(0, rows0, sem0)

        def body(i, carry):
            c0 = 2 * i
            c1 = c0 + 1

            @pl.when(c1 < nch)
            def _():
                gstart(c1, rows1, sem1)

            gwait(rows0, sem0)
            wout(c0, rows0)

            @pl.when(c0 + 2 < nch)
            def _():
                gstart(c0 + 2, rows0, sem0)

            @pl.when(c1 < nch)
            def _():
                gwait(rows1, sem1)
                wout(c1, rows1)

            return carry

        lax.fori_loop(0, (nch + 1) // 2, body, 0)

    return gather_k(table, idx_pad)


def _kernel_b(g0_ref, g1_ref, g2_ref, g3_ref, h_ref, nd_ref,
              wemb_ref, bemb_ref,
              wnd_ref, bnd_ref, wnnd_ref, bias_ref, wka_ref, wqa_ref, wv_ref,
              wp1_ref, bp1_ref, wq1_ref, bq1_ref,
              wq2a_ref, wq2b_ref, bq2_ref, wp2_ref, bp2_ref,
              q2_out, all_out, p2_out):
    f32 = jnp.float32
    bf16 = jnp.bfloat16
    sig = jax.nn.sigmoid
    g_refs = (g0_ref, g1_ref, g2_ref, g3_ref)
    ndist = nd_ref[...]
    ae = jnp.maximum(
        jnp.dot(h_ref[...].astype(bf16), wemb_ref[...],
                preferred_element_type=f32) + bemb_ref[...], 0.0)
    wnd = wnd_ref[...]
    bnd = bnd_ref[...]
    wnnd = wnnd_ref[...]
    wka = wka_ref[...]
    m = []
    ka = []
    for s in range(_NB):
        nds = sig(ndist[:, s:s + 1] * wnd + bnd)
        c = jnp.dot(nds, wnnd, preferred_element_type=f32)
        g = g_refs[s][...]
        glo = lax.bitcast_convert_type(lax.shift_left(g, 16), f32)
        ghi = lax.bitcast_convert_type(lax.bitwise_and(g, jnp.int32(-65536)), f32)
        gg = jnp.concatenate([glo, ghi], axis=1)
        ms = jnp.maximum(gg + c + bias_ref[s:s + 1, :], 0.0)
        m.append(ms)
        ka.append(jnp.dot(ms, wka, preferred_element_type=f32))
    qa = jnp.dot(ae, wqa_ref[...], preferred_element_type=f32)
    es = []
    for s in range(_NB):
        e = qa + ka[s]
        es.append(jnp.where(e >= 0, e, 0.2 * e))
    emax = jnp.maximum(jnp.maximum(es[0], es[1]), jnp.maximum(es[2], es[3]))
    ps = [jnp.exp(e - emax) for e in es]
    inv = 1.0 / (ps[0] + ps[1] + ps[2] + ps[3])
    attn = [p * inv for p in ps]
    wv = wv_ref[...]
    nemb = None
    for s in range(_NB):
        mv = jnp.dot(m[s].astype(bf16), wv, preferred_element_type=f32)
        arep = jnp.concatenate(
            [jnp.broadcast_to(attn[s][:, hh:hh + 1], (attn[s].shape[0], _HD))
             for hh in range(_NH)], axis=1)
        contrib = arep * mv
        nemb = contrib if nemb is None else nemb + contrib
    all_emb = jnp.concatenate([ae, nemb], axis=1)
    all16 = all_emb.astype(bf16)
    p1 = jnp.maximum(jnp.dot(all16, wp1_ref[...], preferred_element_type=f32)
                     + bp1_ref[...], 0.0)
    q1 = jnp.maximum(jnp.dot(all16, wq1_ref[...], preferred_element_type=f32)
                     + bq1_ref[...], 0.0)
    q2_out[...] = (jnp.dot(p1, wq2a_ref[...], preferred_element_type=f32)
                   + jnp.dot(q1, wq2b_ref[...], preferred_element_type=f32)
                   + bq2_ref[...])
    all_out[...] = all_emb
    p2_out[...] = jnp.dot(p1, wp2_ref[...], preferred_element_type=f32) + bp2_ref[...]


def _row2(n):
    return pl.BlockSpec(n, lambda i: (0,) * len(n))


def kernel(car_num, queue_length, occupancy, flow, stop_car_num, pressure,
           current_phase, mask, neighbor_index, neighbor_dis, hidden_state, params):
    p = params
    f32 = jnp.float32
    bf16 = jnp.bfloat16
    bs = car_num.shape[0]
    grid = bs // _BLK

    wf = jnp.stack([p["w_" + f][0] for f in _FEATS])
    bf = jnp.stack([p["b_" + f] for f in _FEATS])
    wrep = jnp.broadcast_to(wf.T[:, :, None], (4, 6, 7)).reshape(4, 42)
    brep = jnp.broadcast_to(bf.T[:, :, None], (4, 6, 7)).reshape(4, 42)
    w_fc1p = p["w_fc1"][_enc_perm()]
    wn_h = p["w_neigh"][:_H]
    wn_nd = p["w_neigh"][_H:_H + 4]
    bias_slots = p["w_neigh"][_H + 4:_H + 8] + p["b_neigh"][None, :]
    wq_a = (p["w_q"].reshape(_H, _NH, _HD) * p["a_src"][None]).sum(-1)
    wk_a = (p["w_k"].reshape(_H, _NH, _HD) * p["a_dst"][None]).sum(-1)

    ph = current_phase.astype(jnp.int32)
    mk = mask.astype(jnp.int32)

    row = lambda i: (i, 0)
    smem = pl.BlockSpec(memory_space=pltpu.SMEM)
    h, u = pl.pallas_call(
        _kernel_a,
        grid=(grid,),
        in_specs=[
            pl.BlockSpec((_BLK, _L), row)] * 6 + [
            pl.BlockSpec((_BLK, _L), row),
            pl.BlockSpec((_BLK, _L), row),
            pl.BlockSpec((_BLK, _H), row),
            _row2((4, 42)), _row2((4, 42)), smem, smem,
            _row2((224, _H)), _row2((1, _H)),
            _row2((_H, 3 * _H)), _row2((1, 3 * _H)),
            _row2((_H, 3 * _H)), _row2((1, 3 * _H)),
            _row2((_H, _H)),
        ],
        out_specs=[pl.BlockSpec((_BLK, _H), row),
                   pl.BlockSpec((_BLK, 128), row)],
        out_shape=[jax.ShapeDtypeStruct((bs, _H), f32),
                   jax.ShapeDtypeStruct((bs, 128), jnp.int32)],
        compiler_params=pltpu.CompilerParams(
            dimension_semantics=("arbitrary",)),
    )(car_num, queue_length, occupancy, flow, stop_car_num, pressure,
      ph, mk, hidden_state,
      wrep, brep, p["phase_emb"], p["mask_emb"],
      w_fc1p.astype(bf16), p["b_fc1"][None, :],
      p["w_ih"].astype(bf16), p["b_ih"][None, :],
      p["w_hh"].astype(bf16), p["b_hh"][None, :],
      wn_h.astype(bf16))

    nbr = neighbor_index.astype(jnp.int32)
    weights_b = (
        p["w_emb"].astype(bf16), p["b_emb"][None, :],
        p["w_nd"], p["b_nd"][None, :], wn_nd, bias_slots, wk_a, wq_a,
        p["w_v"].astype(bf16),
        p["w_p1"].astype(bf16), p["b_p1"][None, :],
        p["w_q1"].astype(bf16), p["b_q1"][None, :],
        p["w_q2"][:_H], p["w_q2"][_H:], p["b_q2"][None, :],
        p["w_p2"], p["b_p2"][None, :])
    wspecs = [
        _row2((_H, _H)), _row2((1, _H)),
        _row2((1, 4)), _row2((1, 4)), _row2((4, _H)), _row2((4, _H)),
        _row2((_H, 4)), _row2((_H, 4)), _row2((_H, _H)),
        _row2((2 * _H, _H)), _row2((1, _H)),
        _row2((2 * _H, _H)), _row2((1, _H)),
        _row2((_H, 8)), _row2((_H, 8)), _row2((1, 8)),
        _row2((_H, 8)), _row2((1, 8)),
    ]

    def run_b(gath, rows, blk0):
        nblk = rows // _BLK
        spb = rows // _BLK
        return pl.pallas_call(
            _kernel_b,
            grid=(nblk,),
            in_specs=[
                pl.BlockSpec((_BLK, 128), lambda i, s=s: (s * spb + i, 0))
                for s in range(_NB)] + [
                pl.BlockSpec((_BLK, _H), lambda i: (blk0 + i, 0)),
                pl.BlockSpec((_BLK, _NB), lambda i: (blk0 + i, 0)),
            ] + wspecs,
            out_specs=[
                pl.BlockSpec((_BLK, 8), row),
                pl.BlockSpec((_BLK, 2 * _H), row),
                pl.BlockSpec((_BLK, 8), row),
            ],
            out_shape=[
                jax.ShapeDtypeStruct((rows, 8), f32),
                jax.ShapeDtypeStruct((rows, 2 * _H), f32),
                jax.ShapeDtypeStruct((rows, 8), f32),
            ],
            compiler_params=pltpu.CompilerParams(
                dimension_semantics=("arbitrary",)),
        )(gath, gath, gath, gath, h, neighbor_dis, *weights_b)

    halves = []
    blk0 = 0
    for rows in (_ROWS1, _ROWS2):
        n_idx = rows * _NB
        per_w = ((n_idx + _NW * _CHUNK - 1) // (_NW * _CHUNK)) * _CHUNK
        idx_flat = nbr[blk0 * _BLK:blk0 * _BLK + rows].T.reshape(-1)
        idx_pad = jnp.concatenate(
            [idx_flat, jnp.zeros((_NW * per_w - n_idx,), jnp.int32)])
        gath = _sc_gather(u, idx_pad, per_w)
        halves.append((gath, rows, blk0))
        blk0 += rows // _BLK

    outs = [run_b(*hv) for hv in halves]
    q2 = jnp.concatenate([outs[0][0], outs[1][0]])
    all_emb = jnp.concatenate([outs[0][1], outs[1][1]])
    p2 = jnp.concatenate([outs[0][2], outs[1][2]])

    return (q2, h, all_emb, p2)


# speedup vs baseline: 1.2079x; 1.2079x over previous
import functools

import jax
import jax.numpy as jnp
import numpy as np
from jax import lax
from jax.experimental import pallas as pl
from jax.experimental.pallas import tpu as pltpu
from jax.experimental.pallas import tpu_sc as plsc

_FEATS = ["car_num", "queue_length", "occupancy", "flow", "stop_car_num", "pressure"]
_BS = 50000
_L = 7
_H = 256
_NB = 4
_NH = 4
_HD = 64
_BLK = 2000

_NW = 32
_CHUNK = 128
_PER_W = 6272
_PAD_N = _NW * _PER_W


def _enc_perm() -> np.ndarray:
    perm = np.empty(224, np.int32)
    for j in range(4):
        for g in range(6):
            for l in range(_L):
                perm[j * 42 + g * 7 + l] = l * 32 + g * 4 + j
    for j in range(4):
        for l in range(_L):
            perm[168 + j * 7 + l] = l * 32 + 24 + j
            perm[196 + j * 7 + l] = l * 32 + 28 + j
    return perm


def _kernel_a(f0_ref, f1_ref, f2_ref, f3_ref, f4_ref, f5_ref,
              ph_ref, mk_ref, hid_ref,
              wrep_ref, brep_ref, pe_ref, me_ref,
              wfc1_ref, bfc1_ref, wih_ref, bih_ref, whh_ref, bhh_ref,
              wnh_ref,
              h_out, u_out):
    f32 = jnp.float32
    sig = jax.nn.sigmoid
    xall = jnp.concatenate(
        [f0_ref[...], f1_ref[...], f2_ref[...], f3_ref[...],
         f4_ref[...], f5_ref[...]], axis=1)
    ph = ph_ref[...]
    mk = mk_ref[...]
    parts = []
    for j in range(4):
        parts.append(xall * wrep_ref[j:j + 1, :] + brep_ref[j:j + 1, :])
    for j in range(4):
        parts.append(jnp.where(ph == 0, pe_ref[0, j], pe_ref[1, j]))
    for j in range(4):
        parts.append(jnp.where(mk == 0, me_ref[0, j], me_ref[1, j]))
    bf16 = jnp.bfloat16
    enc = sig(jnp.concatenate(parts, axis=1))
    x = jnp.maximum(
        jnp.dot(enc.astype(bf16), wfc1_ref[...], preferred_element_type=f32)
        + bfc1_ref[...], 0.0)
    hid = hid_ref[...]
    gi = jnp.dot(x.astype(bf16), wih_ref[...],
                 preferred_element_type=f32) + bih_ref[...]
    gh = jnp.dot(hid.astype(bf16), whh_ref[...],
                 preferred_element_type=f32) + bhh_ref[...]
    r = sig(gi[:, :_H] + gh[:, :_H])
    z = sig(gi[:, _H:2 * _H] + gh[:, _H:2 * _H])
    n = jnp.tanh(gi[:, 2 * _H:] + r * gh[:, 2 * _H:])
    h = (1.0 - z) * n + z * hid
    h16 = h.astype(bf16)
    h_out[...] = h
    u = jnp.dot(h16, wnh_ref[...], preferred_element_type=f32)
    lo = lax.bitcast_convert_type(
        u[:, :128].astype(jnp.bfloat16).astype(f32), jnp.int32)
    hi = lax.bitcast_convert_type(
        u[:, 128:].astype(jnp.bfloat16).astype(f32), jnp.int32)
    u_out[...] = lax.bitwise_or(
        lax.shift_right_logical(lo, 16), lax.bitwise_and(hi, jnp.int32(-65536)))


def _sc_gather(table, idx_pad):
    mesh = plsc.VectorSubcoreMesh(core_axis_name="c", subcore_axis_name="s")

    @functools.partial(
        pl.kernel,
        out_type=jax.ShapeDtypeStruct((_PAD_N, 128), jnp.int32),
        mesh=mesh,
        scratch_types=[
            pltpu.VMEM((_PER_W,), jnp.int32),
            pltpu.VMEM((_CHUNK, 128), jnp.int32),
            pltpu.VMEM((_CHUNK, 128), jnp.int32),
            pltpu.SemaphoreType.DMA,
            pltpu.SemaphoreType.DMA,
        ],
    )
    def gather_k(table_hbm, idx_hbm, out_hbm, idx_v, rows0, rows1, sem0, sem1):
        wid = lax.axis_index("s") * 2 + lax.axis_index("c")
        base = pl.multiple_of(wid * _PER_W, 8)
        pltpu.sync_copy(idx_hbm.at[pl.ds(base, _PER_W)], idx_v)
        nch = _PER_W // _CHUNK

        def gstart(c, buf, sem):
            off = pl.multiple_of(c * _CHUNK, 8)
            pltpu.async_copy(table_hbm.at[idx_v.at[pl.ds(off, _CHUNK)]], buf, sem)

        def gwait(buf, sem):
            pltpu.make_async_copy(
                table_hbm.at[idx_v.at[pl.ds(0, _CHUNK)]], buf, sem).wait()

        def wout(c, buf):
            off = pl.multiple_of(c * _CHUNK, 8)
            pltpu.sync_copy(buf, out_hbm.at[pl.ds(base + off, _CHUNK)])

        gstart(0, rows0, sem0)

        def body(i, carry):
            c0 = 2 * i
            c1 = c0 + 1

            @pl.when(c1 < nch)
            def _():
                gstart(c1, rows1, sem1)

            gwait(rows0, sem0)
            wout(c0, rows0)

            @pl.when(c0 + 2 < nch)
            def _():
                gstart(c0 + 2, rows0, sem0)

            @pl.when(c1 < nch)
            def _():
                gwait(rows1, sem1)
                wout(c1, rows1)

            return carry

        lax.fori_loop(0, (nch + 1) // 2, body, 0)

    return gather_k(table, idx_pad)


def _kernel_b(g0_ref, g1_ref, g2_ref, g3_ref, h_ref, nd_ref,
              wemb_ref, bemb_ref,
              wnd_ref, bnd_ref, wnnd_ref, bias_ref, wka_ref, wqa_ref, wv_ref,
              wp1_ref, bp1_ref, wq1_ref, bq1_ref,
              wq2a_ref, wq2b_ref, bq2_ref, wp2_ref, bp2_ref,
              q2_out, all_out, p2_out):
    f32 = jnp.float32
    bf16 = jnp.bfloat16
    sig = jax.nn.sigmoid
    g_refs = (g0_ref, g1_ref, g2_ref, g3_ref)
    ndist = nd_ref[...]
    ae = jnp.maximum(
        jnp.dot(h_ref[...].astype(bf16), wemb_ref[...],
                preferred_element_type=f32) + bemb_ref[...], 0.0)
    wnd = wnd_ref[...]
    bnd = bnd_ref[...]
    wnnd = wnnd_ref[...]
    wka = wka_ref[...]
    m = []
    ka = []
    for s in range(_NB):
        nds = sig(ndist[:, s:s + 1] * wnd + bnd)
        c = jnp.dot(nds, wnnd, preferred_element_type=f32)
        g = g_refs[s][...]
        glo = lax.bitcast_convert_type(lax.shift_left(g, 16), f32)
        ghi = lax.bitcast_convert_type(lax.bitwise_and(g, jnp.int32(-65536)), f32)
        gg = jnp.concatenate([glo, ghi], axis=1)
        ms = jnp.maximum(gg + c + bias_ref[s:s + 1, :], 0.0)
        m.append(ms)
        ka.append(jnp.dot(ms, wka, preferred_element_type=f32))
    qa = jnp.dot(ae, wqa_ref[...], preferred_element_type=f32)
    es = []
    for s in range(_NB):
        e = qa + ka[s]
        es.append(jnp.where(e >= 0, e, 0.2 * e))
    emax = jnp.maximum(jnp.maximum(es[0], es[1]), jnp.maximum(es[2], es[3]))
    ps = [jnp.exp(e - emax) for e in es]
    inv = 1.0 / (ps[0] + ps[1] + ps[2] + ps[3])
    attn = [p * inv for p in ps]
    wv = wv_ref[...]
    nemb = None
    for s in range(_NB):
        mv = jnp.dot(m[s].astype(bf16), wv, preferred_element_type=f32)
        arep = jnp.concatenate(
            [jnp.broadcast_to(attn[s][:, hh:hh + 1], (attn[s].shape[0], _HD))
             for hh in range(_NH)], axis=1)
        contrib = arep * mv
        nemb = contrib if nemb is None else nemb + contrib
    all_emb = jnp.concatenate([ae, nemb], axis=1)
    all16 = all_emb.astype(bf16)
    p1 = jnp.maximum(jnp.dot(all16, wp1_ref[...], preferred_element_type=f32)
                     + bp1_ref[...], 0.0)
    q1 = jnp.maximum(jnp.dot(all16, wq1_ref[...], preferred_element_type=f32)
                     + bq1_ref[...], 0.0)
    q2_out[...] = (jnp.dot(p1, wq2a_ref[...], preferred_element_type=f32)
                   + jnp.dot(q1, wq2b_ref[...], preferred_element_type=f32)
                   + bq2_ref[...])
    all_out[...] = all_emb
    p2_out[...] = jnp.dot(p1, wp2_ref[...], preferred_element_type=f32) + bp2_ref[...]


def _row2(n):
    return pl.BlockSpec(n, lambda i: (0,) * len(n))


def kernel(car_num, queue_length, occupancy, flow, stop_car_num, pressure,
           current_phase, mask, neighbor_index, neighbor_dis, hidden_state, params):
    p = params
    f32 = jnp.float32
    bf16 = jnp.bfloat16
    bs = car_num.shape[0]
    grid = bs // _BLK

    wf = jnp.stack([p["w_" + f][0] for f in _FEATS])
    bf = jnp.stack([p["b_" + f] for f in _FEATS])
    wrep = jnp.broadcast_to(wf.T[:, :, None], (4, 6, 7)).reshape(4, 42)
    brep = jnp.broadcast_to(bf.T[:, :, None], (4, 6, 7)).reshape(4, 42)
    w_fc1p = p["w_fc1"][_enc_perm()]
    wn_h = p["w_neigh"][:_H]
    wn_nd = p["w_neigh"][_H:_H + 4]
    bias_slots = p["w_neigh"][_H + 4:_H + 8] + p["b_neigh"][None, :]
    wq_a = (p["w_q"].reshape(_H, _NH, _HD) * p["a_src"][None]).sum(-1)
    wk_a = (p["w_k"].reshape(_H, _NH, _HD) * p["a_dst"][None]).sum(-1)

    ph = current_phase.astype(jnp.int32)
    mk = mask.astype(jnp.int32)

    row = lambda i: (i, 0)
    smem = pl.BlockSpec(memory_space=pltpu.SMEM)
    h, u = pl.pallas_call(
        _kernel_a,
        grid=(grid,),
        in_specs=[
            pl.BlockSpec((_BLK, _L), row)] * 6 + [
            pl.BlockSpec((_BLK, _L), row),
            pl.BlockSpec((_BLK, _L), row),
            pl.BlockSpec((_BLK, _H), row),
            _row2((4, 42)), _row2((4, 42)), smem, smem,
            _row2((224, _H)), _row2((1, _H)),
            _row2((_H, 3 * _H)), _row2((1, 3 * _H)),
            _row2((_H, 3 * _H)), _row2((1, 3 * _H)),
            _row2((_H, _H)),
        ],
        out_specs=[pl.BlockSpec((_BLK, _H), row),
                   pl.BlockSpec((_BLK, 128), row)],
        out_shape=[jax.ShapeDtypeStruct((bs, _H), f32),
                   jax.ShapeDtypeStruct((bs, 128), jnp.int32)],
        compiler_params=pltpu.CompilerParams(
            dimension_semantics=("arbitrary",)),
    )(car_num, queue_length, occupancy, flow, stop_car_num, pressure,
      ph, mk, hidden_state,
      wrep, brep, p["phase_emb"], p["mask_emb"],
      w_fc1p.astype(bf16), p["b_fc1"][None, :],
      p["w_ih"].astype(bf16), p["b_ih"][None, :],
      p["w_hh"].astype(bf16), p["b_hh"][None, :],
      wn_h.astype(bf16))

    idx_flat = neighbor_index.astype(jnp.int32).T.reshape(-1)
    idx_pad = jnp.concatenate(
        [idx_flat, jnp.zeros((_PAD_N - idx_flat.shape[0],), jnp.int32)])
    gathered = _sc_gather(u, idx_pad)
    spb = bs // _BLK

    q2, all_emb, p2 = pl.pallas_call(
        _kernel_b,
        grid=(grid,),
        in_specs=[
            pl.BlockSpec((_BLK, 128), lambda i, s=s: (s * spb + i, 0))
            for s in range(_NB)] + [
            pl.BlockSpec((_BLK, _H), row),
            pl.BlockSpec((_BLK, _NB), row),
            _row2((_H, _H)), _row2((1, _H)),
            _row2((1, 4)), _row2((1, 4)), _row2((4, _H)), _row2((4, _H)),
            _row2((_H, 4)), _row2((_H, 4)), _row2((_H, _H)),
            _row2((2 * _H, _H)), _row2((1, _H)),
            _row2((2 * _H, _H)), _row2((1, _H)),
            _row2((_H, 8)), _row2((_H, 8)), _row2((1, 8)),
            _row2((_H, 8)), _row2((1, 8)),
        ],
        out_specs=[
            pl.BlockSpec((_BLK, 8), row),
            pl.BlockSpec((_BLK, 2 * _H), row),
            pl.BlockSpec((_BLK, 8), row),
        ],
        out_shape=[
            jax.ShapeDtypeStruct((bs, 8), f32),
            jax.ShapeDtypeStruct((bs, 2 * _H), f32),
            jax.ShapeDtypeStruct((bs, 8), f32),
        ],
        compiler_params=pltpu.CompilerParams(
            dimension_semantics=("arbitrary",)),
    )(gathered, gathered, gathered, gathered, h, neighbor_dis,
      p["w_emb"].astype(bf16), p["b_emb"][None, :],
      p["w_nd"], p["b_nd"][None, :], wn_nd, bias_slots, wk_a, wq_a,
      p["w_v"].astype(bf16),
      p["w_p1"].astype(bf16), p["b_p1"][None, :],
      p["w_q1"].astype(bf16), p["b_q1"][None, :],
      p["w_q2"][:_H], p["w_q2"][_H:], p["b_q2"][None, :],
      p["w_p2"], p["b_p2"][None, :])

    return (q2, h, all_emb, p2)
